# baseline (speedup 1.0000x reference)
import dataclasses
import functools

import jax
import jax.numpy as jnp
from jax import lax
from jax.experimental import pallas as pl
from jax.experimental.pallas import tpu as pltpu
from jax.experimental.pallas import tpu_sc as plsc

NC = 2
NS = 16
EK = 128
EKS = 192
RK = 128
G_NUM = 1000

f32 = jnp.float32
i32 = jnp.int32


def _ceil_to(x, m):
  return (x + m - 1) // m * m


def _vmesh():
  return plsc.VectorSubcoreMesh(core_axis_name="c", subcore_axis_name="s")


def _sc_params():
  cp = pltpu.CompilerParams()
  cp = dataclasses.replace(cp, needs_layout_passes=False,
                           use_tc_tiling_on_sc=False)
  return cp


def _shared(rows, cols):
  return pltpu.VMEM_SHARED((rows, cols), f32)


def _sc_prep(embed, cat_pad, dst_p, w_p, z16, n_pad, e_pad):
  de = embed.shape[1]
  cd = e_pad // (NC * NS * EK)
  re = n_pad // (NC * NS)
  ce = re // RK
  rt = n_pad // NS

  @functools.partial(
      pl.kernel,
      out_type=(
          jax.ShapeDtypeStruct((n_pad, de), f32),
          jax.ShapeDtypeStruct((NC, n_pad, 16), f32),
      ),
      mesh=_vmesh(),
      scratch_types=[
          _shared(n_pad, 16),
          pltpu.VMEM((RK,), i32),
          pltpu.VMEM((RK, de), f32),
          pltpu.VMEM((EK,), i32),
          pltpu.VMEM((EK,), f32),
          pltpu.VMEM((EK, 16), f32),
          pltpu.SemaphoreType.DMA,
      ],
      compiler_params=_sc_params(),
  )
  def k(embed_ref, cat_ref, dst_ref, w_ref, z_ref, motif_ref, degp_ref,
        accd, cidx, ebuf, dstv, wv, wbuf, sem):
    c = lax.axis_index("c")
    s = lax.axis_index("s")
    g = c * NS + s
    pltpu.sync_copy(z_ref.at[pl.ds(s * rt, rt)], accd.at[pl.ds(s * rt, rt)])

    @pl.loop(0, ce)
    def _(ch):
      rbase = g * re + ch * RK
      pltpu.sync_copy(cat_ref.at[pl.ds(rbase, RK)], cidx)
      pltpu.async_copy(embed_ref.at[cidx], ebuf, sem).wait()
      pltpu.sync_copy(ebuf, motif_ref.at[pl.ds(rbase, RK)])

    plsc.subcore_barrier()

    @pl.loop(0, cd)
    def _(ch):
      base = (g * cd + ch) * EK
      pltpu.sync_copy(dst_ref.at[pl.ds(base, EK)], dstv)
      pltpu.sync_copy(w_ref.at[pl.ds(base, EK)], wv)

      @pl.loop(0, EK)
      def _(e):
        wb = plsc.load_gather(wv, [jnp.full((16,), e, i32)])
        wbuf[e, pl.ds(0, 16)] = wb

      pltpu.sync_copy(wbuf, accd.at[dstv], add=True)

    plsc.subcore_barrier()
    pltpu.sync_copy(accd.at[pl.ds(s * rt, rt)],
                    degp_ref.at[c, pl.ds(s * rt, rt)])

  return k(embed, cat_pad, dst_p, w_p, z16)


def _sc_spmm(m2, srcoff, dst_p, w_p, n_pad, e_pad):
  cs = e_pad // (NS * EKS)
  rt = n_pad // NS
  assert cs % 3 == 0 and cs >= 6

  @functools.partial(
      pl.kernel,
      out_type=jax.ShapeDtypeStruct((NC, n_pad, 32), f32),
      mesh=_vmesh(),
      scratch_types=[
          _shared(n_pad, 32),
          pltpu.VMEM((EKS,), i32), pltpu.VMEM((EKS,), i32),
          pltpu.VMEM((EKS,), i32),
          pltpu.VMEM((EKS,), i32), pltpu.VMEM((EKS,), i32),
          pltpu.VMEM((EKS,), i32),
          pltpu.VMEM((EKS,), f32), pltpu.VMEM((EKS,), f32),
          pltpu.VMEM((EKS,), f32),
          pltpu.VMEM((EKS, 32), f32), pltpu.VMEM((EKS, 32), f32),
          pltpu.VMEM((EKS, 32), f32),
          pltpu.SemaphoreType.DMA, pltpu.SemaphoreType.DMA,
          pltpu.SemaphoreType.DMA,
          pltpu.SemaphoreType.DMA, pltpu.SemaphoreType.DMA,
          pltpu.SemaphoreType.DMA,
      ],
      compiler_params=_sc_params(),
  )
  def k(m_ref, so_ref, dst_ref, w_ref, zout_ref, acc, *bufs_flat):
    srcvs = bufs_flat[0:3]
    dstvs = bufs_flat[3:6]
    wvs = bufs_flat[6:9]
    gbufs = bufs_flat[9:12]
    gsems = bufs_flat[12:15]
    ssems = bufs_flat[15:18]
    c = lax.axis_index("c")
    s = lax.axis_index("s")
    pltpu.sync_copy(m_ref.at[pl.ds(c * n_pad + s * rt, rt)],
                    acc.at[pl.ds(s * rt, rt)])
    plsc.subcore_barrier()

    def lg(ch, b):
      base = (s * cs + ch) * EKS
      pltpu.sync_copy(so_ref.at[c, pl.ds(base, EKS)], srcvs[b])
      pltpu.sync_copy(dst_ref.at[pl.ds(base, EKS)], dstvs[b])
      pltpu.sync_copy(w_ref.at[pl.ds(base, EKS)], wvs[b])
      pltpu.make_async_copy(m_ref.at[srcvs[b]], gbufs[b], gsems[b]).start()

    def wg(b):
      pltpu.make_async_copy(m_ref.at[srcvs[b]], gbufs[b], gsems[b]).wait()

    def mult(b):
      @pl.loop(0, EKS, step=8)
      def _(e0, wv=wvs[b], gbuf=gbufs[b]):
        for u in range(8):
          e = e0 + u
          wb = plsc.load_gather(wv, [jnp.full((16,), e, i32)])
          gbuf[e, pl.ds(0, 16)] = gbuf[e, pl.ds(0, 16)] * wb
          gbuf[e, pl.ds(16, 16)] = gbuf[e, pl.ds(16, 16)] * wb

    def ss(b):
      pltpu.async_copy(gbufs[b], acc.at[dstvs[b]], ssems[b], add=True)

    def wsc(b):
      pltpu.make_async_copy(gbufs[b], acc.at[dstvs[b]], ssems[b]).wait()

    lg(0, 0)
    lg(1, 1)
    wg(0); mult(0); ss(0); lg(2, 2)
    wg(1); mult(1); ss(1); wsc(0); lg(3, 0)
    wg(2); mult(2); ss(2); wsc(1); lg(4, 1)

    @pl.loop(1, cs // 3)
    def _(i):
      for b in range(3):
        wg(b)
        mult(b)
        ss(b)
        bp = (b + 2) % 3
        wsc(bp)
        lg(lax.rem(3 * i + b + 2, cs), bp)

    wg(0)
    wg(1)
    wsc(2)

    plsc.subcore_barrier()
    pltpu.sync_copy(acc.at[pl.ds(s * rt, rt)],
                    zout_ref.at[c, pl.ds(s * rt, rt)])

  return k(m2, srcoff, dst_p, w_p)


def _tc_layer0(nf_pad, motif, degp, w0, n_pad, n):
  bsz = min(2048, n_pad)
  di = nf_pad.shape[1]
  de = motif.shape[1]

  def body(nf_ref, mo_ref, dg_ref, w_ref, m_ref, dv_ref):
    pid = pl.program_id(0)
    d = dg_ref[...]
    dsum = 1.0 + d[0, :, 0:1] + d[1, :, 0:1]
    rows = lax.broadcasted_iota(i32, (bsz, 1), 0) + pid * bsz
    dv = jnp.where(rows < n, lax.rsqrt(dsum), 0.0)
    w = w_ref[...]
    m = jnp.dot(nf_ref[...], w[:di]) + jnp.dot(mo_ref[...], w[di:])
    m = m * dv
    m_ref[0, :, :] = m[:, :32]
    m_ref[1, :, :] = m[:, 32:]
    dv_ref[...] = dv

  return pl.pallas_call(
      body,
      grid=(n_pad // bsz,),
      in_specs=[
          pl.BlockSpec((bsz, di), lambda i: (i, 0)),
          pl.BlockSpec((bsz, de), lambda i: (i, 0)),
          pl.BlockSpec((NC, bsz, 16), lambda i: (0, i, 0)),
          pl.BlockSpec((di + de, 64), lambda i: (0, 0)),
      ],
      out_specs=[
          pl.BlockSpec((NC, bsz, 32), lambda i: (0, i, 0)),
          pl.BlockSpec((bsz, 1), lambda i: (i, 0)),
      ],
      out_shape=[
          jax.ShapeDtypeStruct((NC, n_pad, 32), f32),
          jax.ShapeDtypeStruct((n_pad, 1), f32),
      ],
  )(nf_pad, motif, degp, w0)


def _tc_mid(z, dinv, bl, wl, n_pad):
  bsz = min(2048, n_pad)

  def body(z_ref, dv_ref, b_ref, w_ref, h_ref, m_ref):
    dv = dv_ref[...]
    x = jnp.concatenate([z_ref[0], z_ref[1]], axis=-1)
    h = jnp.maximum(x * dv + b_ref[...][None, :], 0.0)
    h_ref[...] = h
    m = jnp.dot(h, w_ref[...]) * dv
    m_ref[0, :, :] = m[:, :32]
    m_ref[1, :, :] = m[:, 32:]

  return pl.pallas_call(
      body,
      grid=(n_pad // bsz,),
      in_specs=[
          pl.BlockSpec((NC, bsz, 32), lambda i: (0, i, 0)),
          pl.BlockSpec((bsz, 1), lambda i: (i, 0)),
          pl.BlockSpec((64,), lambda i: (0,)),
          pl.BlockSpec((64, 64), lambda i: (0, 0)),
      ],
      out_specs=[
          pl.BlockSpec((bsz, 64), lambda i: (i, 0)),
          pl.BlockSpec((NC, bsz, 32), lambda i: (0, i, 0)),
      ],
      out_shape=[
          jax.ShapeDtypeStruct((n_pad, 64), f32),
          jax.ShapeDtypeStruct((NC, n_pad, 32), f32),
      ],
  )(z, dinv, bl, wl)


def _tc_last(z, dinv, bl, n_pad):
  bsz = min(2048, n_pad)

  def body(z_ref, dv_ref, b_ref, h_ref):
    x = jnp.concatenate([z_ref[0], z_ref[1]], axis=-1)
    h_ref[...] = jnp.maximum(x * dv_ref[...] + b_ref[...][None, :], 0.0)

  return pl.pallas_call(
      body,
      grid=(n_pad // bsz,),
      in_specs=[
          pl.BlockSpec((NC, bsz, 32), lambda i: (0, i, 0)),
          pl.BlockSpec((bsz, 1), lambda i: (i, 0)),
          pl.BlockSpec((64,), lambda i: (0,)),
      ],
      out_specs=pl.BlockSpec((bsz, 64), lambda i: (i, 0)),
      out_shape=jax.ShapeDtypeStruct((n_pad, 64), f32),
  )(z, dinv, bl)


def _sc_readout(hs, batch_pad, zro, zc, ones16, n_pad, g_pad):
  num_l = len(hs)
  rt = n_pad // (NC * NS)
  cr = rt // RK
  gt = g_pad // NS

  scratch = [_shared(g_pad, 64) for _ in range(num_l)]
  scratch += [
      _shared(g_pad, 16),
      pltpu.VMEM((RK,), i32),
      pltpu.VMEM((RK, 64), f32),
      pltpu.VMEM((RK, 16), f32),
  ]

  @functools.partial(
      pl.kernel,
      out_type=(
          jax.ShapeDtypeStruct((NC, num_l, g_pad, 64), f32),
          jax.ShapeDtypeStruct((NC, g_pad, 16), f32),
      ),
      mesh=_vmesh(),
      scratch_types=scratch,
      compiler_params=_sc_params(),
  )
  def k(*refs):
    h_refs = refs[:num_l]
    b_ref, zro_ref, zc_ref, ones_ref = refs[num_l:num_l + 4]
    sums_ref, cnt_ref = refs[num_l + 4:num_l + 6]
    accs = refs[num_l + 6:num_l + 6 + num_l]
    cntac, bidx, hbuf, obuf = refs[num_l + 6 + num_l:]

    c = lax.axis_index("c")
    s = lax.axis_index("s")
    for li in range(num_l):
      pltpu.sync_copy(zro_ref.at[pl.ds(s * gt, gt)],
                      accs[li].at[pl.ds(s * gt, gt)])
    pltpu.sync_copy(zc_ref.at[pl.ds(s * gt, gt)], cntac.at[pl.ds(s * gt, gt)])
    pltpu.sync_copy(ones_ref, obuf)
    plsc.subcore_barrier()

    for li in range(num_l):
      @pl.loop(0, cr)
      def _(ch, h_ref=h_refs[li], acc_ref=accs[li], with_cnt=(li == 0)):
        rbase = (c * NS + s) * rt + ch * RK
        pltpu.sync_copy(b_ref.at[pl.ds(rbase, RK)], bidx)
        pltpu.sync_copy(h_ref.at[pl.ds(rbase, RK)], hbuf)
        pltpu.sync_copy(hbuf, acc_ref.at[bidx], add=True)
        if with_cnt:
          pltpu.sync_copy(obuf, cntac.at[bidx], add=True)

    plsc.subcore_barrier()
    for li in range(num_l):
      pltpu.sync_copy(accs[li].at[pl.ds(s * gt, gt)],
                      sums_ref.at[c, li, pl.ds(s * gt, gt)])
    pltpu.sync_copy(cntac.at[pl.ds(s * gt, gt)],
                    cnt_ref.at[c, pl.ds(s * gt, gt)])

  return k(*hs, batch_pad, zro, zc, ones16)


def _tc_final(sums, cnt, g_num, g_pad, num_l):
  def body(s_ref, c_ref, o_ref):
    x = s_ref[...]
    y = jnp.concatenate([x[0, 0] + x[1, 0], x[0, 1] + x[1, 1]], axis=-1)
    cn = c_ref[...]
    csum = (cn[0, :, 0:1] + cn[1, :, 0:1])[:g_num]
    o_ref[...] = y[:g_num] / jnp.maximum(csum, 1.0)

  return pl.pallas_call(
      body,
      grid=(num_l // 2,),
      in_specs=[
          pl.BlockSpec((NC, 2, g_pad, 64), lambda j: (0, j, 0, 0)),
          pl.BlockSpec((NC, g_pad, 16), lambda j: (0, 0, 0)),
      ],
      out_specs=pl.BlockSpec((g_num, 128), lambda j: (0, j)),
      out_shape=jax.ShapeDtypeStruct((g_num, 64 * num_l), f32),
  )(sums, cnt)


def kernel(original_graph_node_categorical_features, node_features, edge_index,
           edge_features, batch_index, embed, W0, b0, W, b):
  cat = original_graph_node_categorical_features.astype(i32)
  n, di = node_features.shape
  e = edge_index.shape[1]
  num_l = W.shape[0] + 1
  g_num = G_NUM

  n_pad = _ceil_to(n, 4096)
  e_pad = _ceil_to(e, 36864)
  g_pad = _ceil_to(g_num + 1, NS * 8)

  cat_pad = jnp.concatenate([cat, jnp.zeros((n_pad - n,), i32)])
  nf_pad = jnp.concatenate(
      [node_features, jnp.zeros((n_pad - n, di), f32)], axis=0)
  batch_pad = jnp.concatenate(
      [batch_index.astype(i32), jnp.full((n_pad - n,), g_num, i32)])
  src = edge_index[0].astype(i32)
  dst = edge_index[1].astype(i32)
  pe = e_pad - e
  src_p = jnp.concatenate([src, jnp.zeros((pe,), i32)])
  dst_p = jnp.concatenate([dst, jnp.zeros((pe,), i32)])
  w_p = jnp.concatenate([edge_features, jnp.zeros((pe,), f32)])
  srcoff = jnp.stack([src_p, src_p + n_pad])
  z16 = jnp.zeros((n_pad, 16), f32)
  zro = jnp.zeros((g_pad, 64), f32)
  zc = jnp.zeros((g_pad, 16), f32)
  ones16 = jnp.ones((RK, 16), f32)

  motif, degp = _sc_prep(embed, cat_pad, dst_p, w_p, z16, n_pad, e_pad)
  m2, dinv = _tc_layer0(nf_pad, motif, degp, W0, n_pad, n)

  hs = []
  for l in range(num_l):
    z = _sc_spmm(m2.reshape(NC * n_pad, 32), srcoff, dst_p, w_p, n_pad, e_pad)
    bl = b0 if l == 0 else b[l - 1]
    if l < num_l - 1:
      h, m2 = _tc_mid(z, dinv, bl, W[l], n_pad)
    else:
      h = _tc_last(z, dinv, bl, n_pad)
    hs.append(h)

  sums, cnt = _sc_readout(hs, batch_pad, zro, zc, ones16, n_pad, g_pad)
  return _tc_final(sums, cnt, g_num, g_pad, num_l)


# speedup vs baseline: 1.3437x; 1.3437x over previous
import dataclasses
import functools

import jax
import jax.numpy as jnp
from jax import lax
from jax.experimental import pallas as pl
from jax.experimental.pallas import tpu as pltpu
from jax.experimental.pallas import tpu_sc as plsc

NC = 2
NS = 16
EK = 128
EKS = 256
RK = 128
G_NUM = 1000

f32 = jnp.float32
i32 = jnp.int32


def _ceil_to(x, m):
  return (x + m - 1) // m * m


def _vmesh():
  return plsc.VectorSubcoreMesh(core_axis_name="c", subcore_axis_name="s")


def _sc_params():
  cp = pltpu.CompilerParams()
  cp = dataclasses.replace(cp, needs_layout_passes=False,
                           use_tc_tiling_on_sc=False)
  return cp


def _shared(rows, cols):
  return pltpu.VMEM_SHARED((rows, cols), f32)


def _sc_prep(embed, cat_pad, dst_p, w_p, z16, n_pad, e_pad):
  de = embed.shape[1]
  cd = e_pad // (NC * NS * EK)
  re = n_pad // (NC * NS)
  ce = re // RK
  rt = n_pad // NS

  @functools.partial(
      pl.kernel,
      out_type=(
          jax.ShapeDtypeStruct((n_pad, de), f32),
          jax.ShapeDtypeStruct((NC, n_pad, 16), f32),
      ),
      mesh=_vmesh(),
      scratch_types=[
          _shared(n_pad, 16),
          pltpu.VMEM((RK,), i32),
          pltpu.VMEM((RK, de), f32),
          pltpu.VMEM((EK,), i32),
          pltpu.VMEM((EK,), f32),
          pltpu.VMEM((EK, 16), f32),
          pltpu.SemaphoreType.DMA,
      ],
      compiler_params=_sc_params(),
  )
  def k(embed_ref, cat_ref, dst_ref, w_ref, z_ref, motif_ref, degp_ref,
        accd, cidx, ebuf, dstv, wv, wbuf, sem):
    c = lax.axis_index("c")
    s = lax.axis_index("s")
    g = c * NS + s
    pltpu.sync_copy(z_ref.at[pl.ds(s * rt, rt)], accd.at[pl.ds(s * rt, rt)])

    @pl.loop(0, ce)
    def _(ch):
      rbase = g * re + ch * RK
      pltpu.sync_copy(cat_ref.at[pl.ds(rbase, RK)], cidx)
      pltpu.async_copy(embed_ref.at[cidx], ebuf, sem).wait()
      pltpu.sync_copy(ebuf, motif_ref.at[pl.ds(rbase, RK)])

    plsc.subcore_barrier()

    @pl.loop(0, cd)
    def _(ch):
      base = (g * cd + ch) * EK
      pltpu.sync_copy(dst_ref.at[pl.ds(base, EK)], dstv)
      pltpu.sync_copy(w_ref.at[pl.ds(base, EK)], wv)

      @pl.loop(0, EK)
      def _(e):
        wb = plsc.load_gather(wv, [jnp.full((16,), e, i32)])
        wbuf[e, pl.ds(0, 16)] = wb

      pltpu.sync_copy(wbuf, accd.at[dstv], add=True)

    plsc.subcore_barrier()
    pltpu.sync_copy(accd.at[pl.ds(s * rt, rt)],
                    degp_ref.at[c, pl.ds(s * rt, rt)])

  return k(embed, cat_pad, dst_p, w_p, z16)


def _sc_spmm(m2, srcoff, dst_p, w_p, n_pad, e_pad):
  cs = e_pad // (NS * EKS)
  rt = n_pad // NS
  assert cs % 2 == 0

  @functools.partial(
      pl.kernel,
      out_type=jax.ShapeDtypeStruct((NC, n_pad, 32), f32),
      mesh=_vmesh(),
      scratch_types=[
          _shared(n_pad, 32),
          pltpu.VMEM((EKS,), i32), pltpu.VMEM((EKS,), i32),
          pltpu.VMEM((EKS,), i32), pltpu.VMEM((EKS,), i32),
          pltpu.VMEM((EKS,), f32), pltpu.VMEM((EKS,), f32),
          pltpu.VMEM((EKS, 32), f32), pltpu.VMEM((EKS, 32), f32),
          pltpu.SemaphoreType.DMA, pltpu.SemaphoreType.DMA,
      ],
      compiler_params=_sc_params(),
  )
  def k(m_ref, so_ref, dst_ref, w_ref, zout_ref, acc,
        srcv0, srcv1, dstv0, dstv1, wv0, wv1, gbuf0, gbuf1, sem0, sem1):
    c = lax.axis_index("c")
    s = lax.axis_index("s")
    bufs = ((srcv0, dstv0, wv0, gbuf0, sem0),
            (srcv1, dstv1, wv1, gbuf1, sem1))
    pltpu.sync_copy(m_ref.at[pl.ds(c * n_pad + s * rt, rt)],
                    acc.at[pl.ds(s * rt, rt)])
    plsc.subcore_barrier()

    def load_and_gather(ch, b):
      srcv, dstv, wv, gbuf, sem = bufs[b]
      base = (s * cs + ch) * EKS
      pltpu.sync_copy(so_ref.at[c, pl.ds(base, EKS)], srcv)
      pltpu.sync_copy(dst_ref.at[pl.ds(base, EKS)], dstv)
      pltpu.sync_copy(w_ref.at[pl.ds(base, EKS)], wv)
      pltpu.make_async_copy(m_ref.at[srcv], gbuf, sem).start()

    load_and_gather(0, 0)
    load_and_gather(1, 1)

    @pl.loop(0, cs // 2)
    def _(i):
      for b in range(2):
        srcv, dstv, wv, gbuf, sem = bufs[b]
        cur = 2 * i + b
        pltpu.make_async_copy(m_ref.at[srcv], gbuf, sem).wait()

        @pl.loop(0, EKS, step=16)
        def _(e0, wv=wv, gbuf=gbuf):
          wvec = wv[pl.ds(e0, 16)]
          for u in range(16):
            e = e0 + u
            wb = wvec.at[jnp.full((16,), u, i32)].get(
                mode="promise_in_bounds")
            gbuf[e, pl.ds(0, 16)] = gbuf[e, pl.ds(0, 16)] * wb
            gbuf[e, pl.ds(16, 16)] = gbuf[e, pl.ds(16, 16)] * wb

        pltpu.sync_copy(gbuf, acc.at[dstv], add=True)
        load_and_gather(lax.rem(cur + 2, cs), b)

    for b in range(2):
      srcv, dstv, wv, gbuf, sem = bufs[b]
      pltpu.make_async_copy(m_ref.at[srcv], gbuf, sem).wait()

    plsc.subcore_barrier()
    pltpu.sync_copy(acc.at[pl.ds(s * rt, rt)],
                    zout_ref.at[c, pl.ds(s * rt, rt)])

  return k(m2, srcoff, dst_p, w_p)


def _tc_layer0(nf_pad, motif, degp, w0, n_pad, n):
  bsz = min(2048, n_pad)
  di = nf_pad.shape[1]
  de = motif.shape[1]

  def body(nf_ref, mo_ref, dg_ref, w_ref, m_ref, dv_ref):
    pid = pl.program_id(0)
    d = dg_ref[...]
    dsum = 1.0 + d[0, :, 0:1] + d[1, :, 0:1]
    rows = lax.broadcasted_iota(i32, (bsz, 1), 0) + pid * bsz
    dv = jnp.where(rows < n, lax.rsqrt(dsum), 0.0)
    w = w_ref[...]
    m = jnp.dot(nf_ref[...], w[:di]) + jnp.dot(mo_ref[...], w[di:])
    m = m * dv
    m_ref[0, :, :] = m[:, :32]
    m_ref[1, :, :] = m[:, 32:]
    dv_ref[...] = dv

  return pl.pallas_call(
      body,
      grid=(n_pad // bsz,),
      in_specs=[
          pl.BlockSpec((bsz, di), lambda i: (i, 0)),
          pl.BlockSpec((bsz, de), lambda i: (i, 0)),
          pl.BlockSpec((NC, bsz, 16), lambda i: (0, i, 0)),
          pl.BlockSpec((di + de, 64), lambda i: (0, 0)),
      ],
      out_specs=[
          pl.BlockSpec((NC, bsz, 32), lambda i: (0, i, 0)),
          pl.BlockSpec((bsz, 1), lambda i: (i, 0)),
      ],
      out_shape=[
          jax.ShapeDtypeStruct((NC, n_pad, 32), f32),
          jax.ShapeDtypeStruct((n_pad, 1), f32),
      ],
  )(nf_pad, motif, degp, w0)


def _tc_mid(z, dinv, bl, wl, n_pad):
  bsz = min(2048, n_pad)

  def body(z_ref, dv_ref, b_ref, w_ref, h_ref, m_ref):
    dv = dv_ref[...]
    x = jnp.concatenate([z_ref[0], z_ref[1]], axis=-1)
    h = jnp.maximum(x * dv + b_ref[...][None, :], 0.0)
    h_ref[...] = h
    m = jnp.dot(h, w_ref[...]) * dv
    m_ref[0, :, :] = m[:, :32]
    m_ref[1, :, :] = m[:, 32:]

  return pl.pallas_call(
      body,
      grid=(n_pad // bsz,),
      in_specs=[
          pl.BlockSpec((NC, bsz, 32), lambda i: (0, i, 0)),
          pl.BlockSpec((bsz, 1), lambda i: (i, 0)),
          pl.BlockSpec((64,), lambda i: (0,)),
          pl.BlockSpec((64, 64), lambda i: (0, 0)),
      ],
      out_specs=[
          pl.BlockSpec((bsz, 64), lambda i: (i, 0)),
          pl.BlockSpec((NC, bsz, 32), lambda i: (0, i, 0)),
      ],
      out_shape=[
          jax.ShapeDtypeStruct((n_pad, 64), f32),
          jax.ShapeDtypeStruct((NC, n_pad, 32), f32),
      ],
  )(z, dinv, bl, wl)


def _tc_last(z, dinv, bl, n_pad):
  bsz = min(2048, n_pad)

  def body(z_ref, dv_ref, b_ref, h_ref):
    x = jnp.concatenate([z_ref[0], z_ref[1]], axis=-1)
    h_ref[...] = jnp.maximum(x * dv_ref[...] + b_ref[...][None, :], 0.0)

  return pl.pallas_call(
      body,
      grid=(n_pad // bsz,),
      in_specs=[
          pl.BlockSpec((NC, bsz, 32), lambda i: (0, i, 0)),
          pl.BlockSpec((bsz, 1), lambda i: (i, 0)),
          pl.BlockSpec((64,), lambda i: (0,)),
      ],
      out_specs=pl.BlockSpec((bsz, 64), lambda i: (i, 0)),
      out_shape=jax.ShapeDtypeStruct((n_pad, 64), f32),
  )(z, dinv, bl)


def _sc_readout(hs, batch_pad, zro, zc, ones16, n_pad, g_pad):
  num_l = len(hs)
  rt = n_pad // (NC * NS)
  cr = rt // RK
  gt = g_pad // NS

  scratch = [_shared(g_pad, 64) for _ in range(num_l)]
  scratch += [
      _shared(g_pad, 16),
      pltpu.VMEM((RK,), i32),
      pltpu.VMEM((RK, 64), f32),
      pltpu.VMEM((RK, 16), f32),
  ]

  @functools.partial(
      pl.kernel,
      out_type=(
          jax.ShapeDtypeStruct((NC, num_l, g_pad, 64), f32),
          jax.ShapeDtypeStruct((NC, g_pad, 16), f32),
      ),
      mesh=_vmesh(),
      scratch_types=scratch,
      compiler_params=_sc_params(),
  )
  def k(*refs):
    h_refs = refs[:num_l]
    b_ref, zro_ref, zc_ref, ones_ref = refs[num_l:num_l + 4]
    sums_ref, cnt_ref = refs[num_l + 4:num_l + 6]
    accs = refs[num_l + 6:num_l + 6 + num_l]
    cntac, bidx, hbuf, obuf = refs[num_l + 6 + num_l:]

    c = lax.axis_index("c")
    s = lax.axis_index("s")
    for li in range(num_l):
      pltpu.sync_copy(zro_ref.at[pl.ds(s * gt, gt)],
                      accs[li].at[pl.ds(s * gt, gt)])
    pltpu.sync_copy(zc_ref.at[pl.ds(s * gt, gt)], cntac.at[pl.ds(s * gt, gt)])
    pltpu.sync_copy(ones_ref, obuf)
    plsc.subcore_barrier()

    for li in range(num_l):
      @pl.loop(0, cr)
      def _(ch, h_ref=h_refs[li], acc_ref=accs[li], with_cnt=(li == 0)):
        rbase = (c * NS + s) * rt + ch * RK
        pltpu.sync_copy(b_ref.at[pl.ds(rbase, RK)], bidx)
        pltpu.sync_copy(h_ref.at[pl.ds(rbase, RK)], hbuf)
        pltpu.sync_copy(hbuf, acc_ref.at[bidx], add=True)
        if with_cnt:
          pltpu.sync_copy(obuf, cntac.at[bidx], add=True)

    plsc.subcore_barrier()
    for li in range(num_l):
      pltpu.sync_copy(accs[li].at[pl.ds(s * gt, gt)],
                      sums_ref.at[c, li, pl.ds(s * gt, gt)])
    pltpu.sync_copy(cntac.at[pl.ds(s * gt, gt)],
                    cnt_ref.at[c, pl.ds(s * gt, gt)])

  return k(*hs, batch_pad, zro, zc, ones16)


def _tc_final(sums, cnt, g_num, g_pad, num_l):
  def body(s_ref, c_ref, o_ref):
    x = s_ref[...]
    y = jnp.concatenate([x[0, 0] + x[1, 0], x[0, 1] + x[1, 1]], axis=-1)
    cn = c_ref[...]
    csum = (cn[0, :, 0:1] + cn[1, :, 0:1])[:g_num]
    o_ref[...] = y[:g_num] / jnp.maximum(csum, 1.0)

  return pl.pallas_call(
      body,
      grid=(num_l // 2,),
      in_specs=[
          pl.BlockSpec((NC, 2, g_pad, 64), lambda j: (0, j, 0, 0)),
          pl.BlockSpec((NC, g_pad, 16), lambda j: (0, 0, 0)),
      ],
      out_specs=pl.BlockSpec((g_num, 128), lambda j: (0, j)),
      out_shape=jax.ShapeDtypeStruct((g_num, 64 * num_l), f32),
  )(sums, cnt)


def kernel(original_graph_node_categorical_features, node_features, edge_index,
           edge_features, batch_index, embed, W0, b0, W, b):
  cat = original_graph_node_categorical_features.astype(i32)
  n, di = node_features.shape
  e = edge_index.shape[1]
  num_l = W.shape[0] + 1
  g_num = G_NUM

  n_pad = _ceil_to(n, 4096)
  e_pad = _ceil_to(e, max(2 * NS * EKS, NC * NS * EK))
  g_pad = _ceil_to(g_num + 1, NS * 8)

  cat_pad = jnp.concatenate([cat, jnp.zeros((n_pad - n,), i32)])
  nf_pad = jnp.concatenate(
      [node_features, jnp.zeros((n_pad - n, di), f32)], axis=0)
  batch_pad = jnp.concatenate(
      [batch_index.astype(i32), jnp.full((n_pad - n,), g_num, i32)])
  src = edge_index[0].astype(i32)
  dst = edge_index[1].astype(i32)
  pe = e_pad - e
  src_p = jnp.concatenate([src, jnp.zeros((pe,), i32)])
  dst_p = jnp.concatenate([dst, jnp.zeros((pe,), i32)])
  w_p = jnp.concatenate([edge_features, jnp.zeros((pe,), f32)])
  srcoff = jnp.stack([src_p, src_p + n_pad])
  z16 = jnp.zeros((n_pad, 16), f32)
  zro = jnp.zeros((g_pad, 64), f32)
  zc = jnp.zeros((g_pad, 16), f32)
  ones16 = jnp.ones((RK, 16), f32)

  motif, degp = _sc_prep(embed, cat_pad, dst_p, w_p, z16, n_pad, e_pad)
  m2, dinv = _tc_layer0(nf_pad, motif, degp, W0, n_pad, n)

  hs = []
  for l in range(num_l):
    z = _sc_spmm(m2.reshape(NC * n_pad, 32), srcoff, dst_p, w_p, n_pad, e_pad)
    bl = b0 if l == 0 else b[l - 1]
    if l < num_l - 1:
      h, m2 = _tc_mid(z, dinv, bl, W[l], n_pad)
    else:
      h = _tc_last(z, dinv, bl, n_pad)
    hs.append(h)

  sums, cnt = _sc_readout(hs, batch_pad, zro, zc, ones16, n_pad, g_pad)
  return _tc_final(sums, cnt, g_num, g_pad, num_l)
